# Initial kernel scaffold; baseline (speedup 1.0000x reference)
import functools
import math

import jax
import jax.numpy as jnp
from jax import lax
from jax.experimental import pallas as pl
from jax.experimental.pallas import tpu as pltpu
from jax.experimental.pallas import tpu_sc as plsc

N = 10000
E = 320000
D = 128

NC = 2
NS = 16
NW = NC * NS

CHUNK = 128
EDGES_PER_TILE = 10112
NCHUNK = EDGES_PER_TILE // CHUNK
E_PAD = EDGES_PER_TILE * NW
N_ACC = 10016
ROWS_PER_TILE = N_ACC // NS


def _linear_body(x_ref, wt_ref, b_ref, es_ref, o_ref):
    x = x_ref[...]
    h = jnp.dot(x, wt_ref[...], preferred_element_type=jnp.float32) + b_ref[...]
    time = jax.nn.sigmoid(h[:, 0:1]) * es_ref[0, 0] + 1.1
    col = lax.broadcasted_iota(jnp.int32, h.shape, 1)
    narrow = jnp.where(col == 0, 0.0, h)
    sq = jnp.clip(jnp.sum(narrow * narrow, axis=-1, keepdims=True), 1e-8, None)
    root = jnp.sqrt((time * time - 1.0) / sq)
    o_ref[...] = jnp.where(col == 0, time, h * root)


def _lorentz_linear(x, Wt, b2, es):
    blk = 1000
    return pl.pallas_call(
        _linear_body,
        grid=(N // blk,),
        in_specs=[
            pl.BlockSpec((blk, D), lambda i: (i, 0)),
            pl.BlockSpec((D, D), lambda i: (0, 0)),
            pl.BlockSpec((1, D), lambda i: (0, 0)),
            pl.BlockSpec((1, 1), lambda i: (0, 0), memory_space=pltpu.SMEM),
        ],
        out_specs=pl.BlockSpec((blk, D), lambda i: (i, 0)),
        out_shape=jax.ShapeDtypeStruct((N, D), jnp.float32),
    )(x, Wt, b2, es)


def _scatter_body(h_hbm, src_hbm, dst_hbm, zero_hbm, out_hbm,
                  src_v, dst_v, rows_v, stage_v, acc_sh, sem):
    cid = lax.axis_index("c")
    sid = lax.axis_index("s")
    wid = cid * NS + sid

    row0 = sid * ROWS_PER_TILE
    pltpu.sync_copy(zero_hbm.at[pl.ds(0, ROWS_PER_TILE)],
                    acc_sh.at[pl.ds(row0, ROWS_PER_TILE)])

    pltpu.sync_copy(src_hbm.at[wid], src_v)
    pltpu.sync_copy(dst_hbm.at[wid], dst_v)
    plsc.subcore_barrier()

    def chunk_step(j, carry):
        pltpu.async_copy(h_hbm.at[src_v.at[j]], rows_v, sem).wait()
        pltpu.sync_copy(rows_v, acc_sh.at[dst_v.at[j]], add=True)
        return carry

    lax.fori_loop(0, NCHUNK, chunk_step, 0)
    plsc.subcore_barrier()

    pltpu.sync_copy(acc_sh.at[pl.ds(row0, ROWS_PER_TILE)], stage_v)
    pltpu.sync_copy(stage_v, out_hbm.at[cid, pl.ds(row0, ROWS_PER_TILE)])


def _segment_sum(h, src2d, dst2d, zeros):
    mesh = plsc.VectorSubcoreMesh(core_axis_name="c", subcore_axis_name="s")
    k = pl.kernel(
        _scatter_body,
        mesh=mesh,
        out_type=jax.ShapeDtypeStruct((NC, N_ACC, D), jnp.float32),
        scratch_types=[
            pltpu.VMEM((NCHUNK, CHUNK), jnp.int32),
            pltpu.VMEM((NCHUNK, CHUNK), jnp.int32),
            pltpu.VMEM((CHUNK, D), jnp.float32),
            pltpu.VMEM((ROWS_PER_TILE, D), jnp.float32),
            pltpu.VMEM_SHARED((N_ACC, D), jnp.float32),
            pltpu.SemaphoreType.DMA,
        ],
    )
    return k(h, src2d, dst2d, zeros)


def _finalize_body(p_ref, o_ref):
    s = p_ref[0] + p_ref[1]
    s0 = s[:, 0:1]
    sumsq = jnp.sum(s * s, axis=-1, keepdims=True)
    ninner = 2.0 * s0 * s0 - sumsq
    denom = jnp.sqrt(jnp.clip(jnp.abs(ninner), 1e-6, None))
    o_ref[...] = s / denom


def _finalize(partials):
    blk = 1000
    return pl.pallas_call(
        _finalize_body,
        grid=(N // blk,),
        in_specs=[pl.BlockSpec((NC, blk, D), lambda i: (0, i, 0))],
        out_specs=pl.BlockSpec((blk, D), lambda i: (i, 0)),
        out_shape=jax.ShapeDtypeStruct((N, D), jnp.float32),
    )(partials)


def kernel(x, edge_index, W, b, scale_lin):
    Wt = W.T
    b2 = b.reshape(1, D)
    es = jnp.exp(scale_lin).reshape(1, 1)
    h = _lorentz_linear(x, Wt, b2, es)

    src = edge_index[0]
    dst = edge_index[1]
    pad = E_PAD - E
    src_p = jnp.concatenate([src, jnp.zeros((pad,), jnp.int32)])
    dst_p = jnp.concatenate([dst, jnp.full((pad,), N, jnp.int32)])
    src2d = src_p.reshape(NW, NCHUNK, CHUNK)
    dst2d = dst_p.reshape(NW, NCHUNK, CHUNK)
    zeros = jnp.zeros((ROWS_PER_TILE, D), jnp.float32)

    partials = _segment_sum(h, src2d, dst2d, zeros)
    return _finalize(partials)


# speedup vs baseline: 4.4026x; 4.4026x over previous
import functools
import math

import jax
import jax.numpy as jnp
from jax import lax
from jax.experimental import pallas as pl
from jax.experimental.pallas import tpu as pltpu
from jax.experimental.pallas import tpu_sc as plsc

N = 10000
E = 320000
D = 128

NC = 2
NS = 16
NW = NC * NS

CHUNK = 128
EDGES_PER_TILE = 10112
NCHUNK = EDGES_PER_TILE // CHUNK
E_PAD = EDGES_PER_TILE * NW
N_ACC = 10112
ROWS_PER_TILE = N_ACC // NS


def _linear_body(x_ref, wt_ref, b_ref, es_ref, o_ref):
    x = x_ref[...]
    h = jnp.dot(x, wt_ref[...], preferred_element_type=jnp.float32) + b_ref[...]
    time = jax.nn.sigmoid(h[:, 0:1]) * es_ref[0, 0] + 1.1
    col = lax.broadcasted_iota(jnp.int32, h.shape, 1)
    narrow = jnp.where(col == 0, 0.0, h)
    sq = jnp.clip(jnp.sum(narrow * narrow, axis=-1, keepdims=True), 1e-8, None)
    root = jnp.sqrt((time * time - 1.0) / sq)
    o_ref[...] = jnp.where(col == 0, time, h * root)


def _lorentz_linear(x, Wt, b2, es):
    blk = 1000
    return pl.pallas_call(
        _linear_body,
        grid=(N // blk,),
        in_specs=[
            pl.BlockSpec((blk, D), lambda i: (i, 0)),
            pl.BlockSpec((D, D), lambda i: (0, 0)),
            pl.BlockSpec((1, D), lambda i: (0, 0)),
            pl.BlockSpec((1, 1), lambda i: (0, 0), memory_space=pltpu.SMEM),
        ],
        out_specs=pl.BlockSpec((blk, D), lambda i: (i, 0)),
        out_shape=jax.ShapeDtypeStruct((N, D), jnp.float32),
    )(x, Wt, b2, es)


def _scatter_body(h_hbm, src_hbm, dst_hbm, zero_hbm, out_hbm,
                  src_v, dst_v, rows_v, acc_sh, sem):
    cid = lax.axis_index("c")
    sid = lax.axis_index("s")
    wid = cid * NS + sid

    row0 = sid * ROWS_PER_TILE
    pltpu.sync_copy(zero_hbm.at[pl.ds(0, ROWS_PER_TILE)],
                    acc_sh.at[pl.ds(row0, ROWS_PER_TILE)])

    pltpu.sync_copy(src_hbm.at[wid], src_v)
    pltpu.sync_copy(dst_hbm.at[wid], dst_v)
    plsc.subcore_barrier()

    def chunk_step(j, carry):
        pltpu.async_copy(h_hbm.at[src_v.at[j]], rows_v, sem).wait()
        pltpu.sync_copy(rows_v, acc_sh.at[dst_v.at[j]], add=True)
        return carry

    lax.fori_loop(0, NCHUNK, chunk_step, 0)
    plsc.subcore_barrier()

    pltpu.sync_copy(acc_sh.at[pl.ds(row0, ROWS_PER_TILE)],
                    out_hbm.at[cid, pl.ds(row0, ROWS_PER_TILE)])


def _segment_sum(h, src2d, dst2d, zeros):
    mesh = plsc.VectorSubcoreMesh(core_axis_name="c", subcore_axis_name="s")
    k = pl.kernel(
        _scatter_body,
        mesh=mesh,
        out_type=jax.ShapeDtypeStruct((NC, N_ACC, D), jnp.float32),
        scratch_types=[
            pltpu.VMEM((NCHUNK, CHUNK), jnp.int32),
            pltpu.VMEM((NCHUNK, CHUNK), jnp.int32),
            pltpu.VMEM((CHUNK, D), jnp.float32),
            pltpu.VMEM_SHARED((N_ACC, D), jnp.float32),
            pltpu.SemaphoreType.DMA,
        ],
    )
    return k(h, src2d, dst2d, zeros)


def _finalize_body(p_ref, o_ref):
    s = p_ref[0] + p_ref[1]
    s0 = s[:, 0:1]
    sumsq = jnp.sum(s * s, axis=-1, keepdims=True)
    ninner = 2.0 * s0 * s0 - sumsq
    denom = jnp.sqrt(jnp.clip(jnp.abs(ninner), 1e-6, None))
    o_ref[...] = s / denom


def _finalize(partials):
    blk = 1000
    return pl.pallas_call(
        _finalize_body,
        grid=(N // blk,),
        in_specs=[pl.BlockSpec((NC, blk, D), lambda i: (0, i, 0))],
        out_specs=pl.BlockSpec((blk, D), lambda i: (i, 0)),
        out_shape=jax.ShapeDtypeStruct((N, D), jnp.float32),
    )(partials)


def kernel(x, edge_index, W, b, scale_lin):
    Wt = W.T
    b2 = b.reshape(1, D)
    es = jnp.exp(scale_lin).reshape(1, 1)
    h = _lorentz_linear(x, Wt, b2, es)

    src = edge_index[0]
    dst = edge_index[1]
    pad = E_PAD - E
    src_p = jnp.concatenate([src, jnp.zeros((pad,), jnp.int32)])
    dst_p = jnp.concatenate([dst, jnp.full((pad,), N, jnp.int32)])
    src2d = src_p.reshape(NW, NCHUNK, CHUNK)
    dst2d = dst_p.reshape(NW, NCHUNK, CHUNK)
    zeros = jnp.zeros((ROWS_PER_TILE, D), jnp.float32)

    partials = _segment_sum(h, src2d, dst2d, zeros)
    return _finalize(partials)


# speedup vs baseline: 4.4189x; 1.0037x over previous
import functools
import math

import jax
import jax.numpy as jnp
from jax import lax
from jax.experimental import pallas as pl
from jax.experimental.pallas import tpu as pltpu
from jax.experimental.pallas import tpu_sc as plsc

N = 10000
E = 320000
D = 128

NC = 2
NS = 16
NW = NC * NS

CHUNK = 128
NCHUNK = 79
EDGES_PER_TILE = CHUNK * NCHUNK
E_PAD = EDGES_PER_TILE * NW
N_ACC = 10112
ROWS_PER_TILE = N_ACC // NS


def _linear_body(x_ref, wt_ref, b_ref, es_ref, o_ref):
    x = x_ref[...]
    h = jnp.dot(x, wt_ref[...], preferred_element_type=jnp.float32) + b_ref[...]
    time = jax.nn.sigmoid(h[:, 0:1]) * es_ref[0, 0] + 1.1
    col = lax.broadcasted_iota(jnp.int32, h.shape, 1)
    narrow = jnp.where(col == 0, 0.0, h)
    sq = jnp.clip(jnp.sum(narrow * narrow, axis=-1, keepdims=True), 1e-8, None)
    root = jnp.sqrt((time * time - 1.0) / sq)
    o_ref[...] = jnp.where(col == 0, time, h * root)


def _lorentz_linear(x, Wt, b2, es):
    blk = 1000
    return pl.pallas_call(
        _linear_body,
        grid=(N // blk,),
        in_specs=[
            pl.BlockSpec((blk, D), lambda i: (i, 0)),
            pl.BlockSpec((D, D), lambda i: (0, 0)),
            pl.BlockSpec((1, D), lambda i: (0, 0)),
            pl.BlockSpec((1, 1), lambda i: (0, 0), memory_space=pltpu.SMEM),
        ],
        out_specs=pl.BlockSpec((blk, D), lambda i: (i, 0)),
        out_shape=jax.ShapeDtypeStruct((N, D), jnp.float32),
    )(x, Wt, b2, es)


def _scatter_body(h_hbm, src_hbm, dst_hbm, zero_hbm, out_hbm,
                  src_v, dst_v, rows_v, acc_sh, sem):
    cid = lax.axis_index("c")
    sid = lax.axis_index("s")
    wid = cid * NS + sid

    row0 = sid * ROWS_PER_TILE
    pltpu.sync_copy(zero_hbm.at[pl.ds(0, ROWS_PER_TILE)],
                    acc_sh.at[pl.ds(row0, ROWS_PER_TILE)])

    pltpu.sync_copy(src_hbm.at[wid], src_v)
    pltpu.sync_copy(dst_hbm.at[wid], dst_v)
    plsc.subcore_barrier()

    def chunk_step(j, carry):
        pltpu.async_copy(h_hbm.at[src_v.at[j]], rows_v, sem).wait()
        pltpu.sync_copy(rows_v, acc_sh.at[dst_v.at[j]], add=True)
        return carry

    lax.fori_loop(0, NCHUNK, chunk_step, 0)
    plsc.subcore_barrier()

    pltpu.sync_copy(acc_sh.at[pl.ds(row0, ROWS_PER_TILE)],
                    out_hbm.at[cid, pl.ds(row0, ROWS_PER_TILE)])


def _segment_sum(h, src2d, dst2d, zeros):
    mesh = plsc.VectorSubcoreMesh(core_axis_name="c", subcore_axis_name="s")
    k = pl.kernel(
        _scatter_body,
        mesh=mesh,
        out_type=jax.ShapeDtypeStruct((NC, N_ACC, D), jnp.float32),
        scratch_types=[
            pltpu.VMEM((NCHUNK, CHUNK), jnp.int32),
            pltpu.VMEM((NCHUNK, CHUNK), jnp.int32),
            pltpu.VMEM((CHUNK, D), jnp.float32),
            pltpu.VMEM_SHARED((N_ACC, D), jnp.float32),
            pltpu.SemaphoreType.DMA,
        ],
    )
    return k(h, src2d, dst2d, zeros)


def _finalize_body(p_ref, o_ref):
    s = p_ref[0] + p_ref[1]
    s0 = s[:, 0:1]
    sumsq = jnp.sum(s * s, axis=-1, keepdims=True)
    ninner = 2.0 * s0 * s0 - sumsq
    denom = jnp.sqrt(jnp.clip(jnp.abs(ninner), 1e-6, None))
    o_ref[...] = s / denom


def _finalize(partials):
    blk = 1000
    return pl.pallas_call(
        _finalize_body,
        grid=(N // blk,),
        in_specs=[pl.BlockSpec((NC, blk, D), lambda i: (0, i, 0))],
        out_specs=pl.BlockSpec((blk, D), lambda i: (i, 0)),
        out_shape=jax.ShapeDtypeStruct((N, D), jnp.float32),
    )(partials)


def kernel(x, edge_index, W, b, scale_lin):
    Wt = W.T
    b2 = b.reshape(1, D)
    es = jnp.exp(scale_lin).reshape(1, 1)
    h = _lorentz_linear(x, Wt, b2, es)

    src = edge_index[0]
    dst = edge_index[1]
    pad = E_PAD - E
    src_p = jnp.concatenate([src, jnp.zeros((pad,), jnp.int32)])
    dst_p = jnp.concatenate([dst, jnp.full((pad,), N, jnp.int32)])
    src2d = src_p.reshape(NW, NCHUNK, CHUNK)
    dst2d = dst_p.reshape(NW, NCHUNK, CHUNK)
    zeros = jnp.zeros((ROWS_PER_TILE, D), jnp.float32)

    partials = _segment_sum(h, src2d, dst2d, zeros)
    return _finalize(partials)
